# Initial kernel scaffold; baseline (speedup 1.0000x reference)
import jax
import jax.numpy as jnp
from jax.experimental import pallas as pl


def kernel(feats, edges, batch, W1, b1, W2, b2, Wgc, bgc, Wc, bc):
    raise NotImplementedError("write your pallas kernel here")


# speedup vs baseline: 17.5776x; 17.5776x over previous
import jax
import jax.numpy as jnp
from jax import lax
from jax.experimental import pallas as pl
from jax.experimental.pallas import tpu as pltpu
from jax.experimental.pallas import tpu_sc as plsc

N = 10000
E = 320000
D_IN = 128
H1 = 256
H2 = 128
GC_OUT = 128

NC = 2
NS = 16
LANES = 16

CHUNK = 128
NCHUNKS = E // CHUNK
CHUNKS_PER_CORE = NCHUNKS // NC
NP = 10240
ROWS_PER_TILE = NP // NS
RCHUNK = 128

BN = 400
GRID = N // BN


def _sc_mesh():
    return plsc.VectorSubcoreMesh(core_axis_name="c", subcore_axis_name="s",
                                  num_cores=NC, num_subcores=NS)


def _deg_body(dst_hbm, out_hbm, ebuf, degbuf):
    c = lax.axis_index("c")
    s = lax.axis_index("s")
    w = s * NC + c

    zeros16 = jnp.zeros((LANES,), jnp.float32)
    zeros16i = jnp.zeros((LANES,), jnp.int32)
    ones16 = jnp.ones((LANES,), jnp.float32)

    def zloop(i, carry):
        degbuf[0, pl.ds(i * LANES, LANES)] = zeros16
        return carry

    lax.fori_loop(0, N // LANES, zloop, 0)

    nw = NC * NS
    iters = (NCHUNKS + nw - 1) // nw

    def eloop(k, carry):
        ch = w + nw * k

        @pl.when(ch < NCHUNKS)
        def _():
            pltpu.sync_copy(dst_hbm.at[pl.ds(ch * CHUNK, CHUNK)], ebuf)
            for j in range(CHUNK // LANES):
                idx = ebuf[pl.ds(j * LANES, LANES)]
                plsc.addupdate_scatter(degbuf, [zeros16i, idx], ones16)

        return carry

    lax.fori_loop(0, iters, eloop, 0)
    pltpu.sync_copy(degbuf, out_hbm.at[w])


def _make_deg_call():
    return pl.kernel(
        _deg_body,
        out_type=jax.ShapeDtypeStruct((NC * NS, 1, N), jnp.float32),
        mesh=_sc_mesh(),
        scratch_types=[
            pltpu.VMEM((CHUNK,), jnp.int32),
            pltpu.VMEM((1, N), jnp.float32),
        ],
        compiler_params=pltpu.CompilerParams(needs_layout_passes=False),
    )


def _agg_body(src_hbm, dst_hbm, y_hbm, out0_hbm, out1_hbm,
              sidx, didx, rows, acc, sem):
    c = lax.axis_index("c")
    s = lax.axis_index("s")
    r0 = s * ROWS_PER_TILE

    for i in range(ROWS_PER_TILE // RCHUNK):
        pltpu.sync_copy(y_hbm.at[pl.ds(r0 + i * RCHUNK, RCHUNK)], rows)
        pltpu.sync_copy(rows, acc.at[pl.ds(r0 + i * RCHUNK, RCHUNK)])
    plsc.subcore_barrier()

    ch_lo = c * CHUNKS_PER_CORE
    ch_hi = ch_lo + CHUNKS_PER_CORE
    iters = (CHUNKS_PER_CORE + NS - 1) // NS

    def eloop(k, carry):
        ch = ch_lo + s + NS * k

        @pl.when(ch < ch_hi)
        def _():
            pltpu.sync_copy(src_hbm.at[pl.ds(ch * CHUNK, CHUNK)], sidx.at[0])
            pltpu.sync_copy(dst_hbm.at[pl.ds(ch * CHUNK, CHUNK)], didx.at[0])
            pltpu.async_copy(y_hbm.at[sidx.at[0]], rows, sem).wait()
            pltpu.sync_copy(rows, acc.at[didx.at[0]], add=True)

        return carry

    lax.fori_loop(0, iters, eloop, 0)
    plsc.subcore_barrier()

    def writeback(out_hbm):
        for i in range(ROWS_PER_TILE // RCHUNK):
            pltpu.sync_copy(acc.at[pl.ds(r0 + i * RCHUNK, RCHUNK)], rows)
            pltpu.sync_copy(rows, out_hbm.at[pl.ds(r0 + i * RCHUNK, RCHUNK)])

    @pl.when(c == 0)
    def _():
        writeback(out0_hbm)

    @pl.when(c == 1)
    def _():
        writeback(out1_hbm)


def _make_agg_call():
    return pl.kernel(
        _agg_body,
        out_type=(
            jax.ShapeDtypeStruct((NP, GC_OUT), jnp.float32),
            jax.ShapeDtypeStruct((NP, GC_OUT), jnp.float32),
        ),
        mesh=_sc_mesh(),
        scratch_types=[
            pltpu.VMEM((1, CHUNK), jnp.int32),
            pltpu.VMEM((1, CHUNK), jnp.int32),
            pltpu.VMEM((CHUNK, GC_OUT), jnp.float32),
            pltpu.VMEM_SHARED((NP, GC_OUT), jnp.float32),
            pltpu.SemaphoreType.DMA,
        ],
    )


def _dis_body(degp_ref, dis_ref):
    i = pl.program_id(0)
    part = jnp.sum(degp_ref[...], axis=(0, 1))[:, None]

    @pl.when(i == 0)
    def _():
        dis_ref[...] = part

    @pl.when(i > 0)
    def _():
        dis_ref[...] += part

    @pl.when(i == pl.num_programs(0) - 1)
    def _():
        dis_ref[...] = lax.rsqrt(1.0 + dis_ref[...])


def _make_dis_call():
    return pl.pallas_call(
        _dis_body,
        grid=(4,),
        in_specs=[pl.BlockSpec((8, 1, N), lambda i: (i, 0, 0))],
        out_specs=pl.BlockSpec((N, 1), lambda i: (0, 0)),
        out_shape=jax.ShapeDtypeStruct((N, 1), jnp.float32),
    )


def _elu(x):
    return jnp.where(x > 0, x, jnp.exp(x) - 1.0)


def _dot(a, b):
    return jnp.dot(a, b, precision=lax.Precision.HIGHEST,
                   preferred_element_type=jnp.float32)


def _mlp_body(feats_ref, dis_ref, W1_ref, b1_ref, W2_ref, b2_ref, Wgc_ref,
              nf_ref, y_ref):
    x = feats_ref[...]
    h = _elu(_dot(x, W1_ref[...]) + b1_ref[...])
    nf = _elu(_dot(h, W2_ref[...]) + b2_ref[...])
    nf_ref[...] = nf
    xl = _dot(nf, Wgc_ref[0:H2, :]) + _dot(x, Wgc_ref[H2:H2 + D_IN, :])
    y_ref[...] = xl * dis_ref[...]


def _make_mlp_call():
    return pl.pallas_call(
        _mlp_body,
        grid=(GRID,),
        in_specs=[
            pl.BlockSpec((BN, D_IN), lambda i: (i, 0)),
            pl.BlockSpec((BN, 1), lambda i: (i, 0)),
            pl.BlockSpec((D_IN, H1), lambda i: (0, 0)),
            pl.BlockSpec((1, H1), lambda i: (0, 0)),
            pl.BlockSpec((H1, H2), lambda i: (0, 0)),
            pl.BlockSpec((1, H2), lambda i: (0, 0)),
            pl.BlockSpec((H2 + D_IN, GC_OUT), lambda i: (0, 0)),
        ],
        out_specs=[
            pl.BlockSpec((BN, H2), lambda i: (i, 0)),
            pl.BlockSpec((BN, GC_OUT), lambda i: (i, 0)),
        ],
        out_shape=[
            jax.ShapeDtypeStruct((N, H2), jnp.float32),
            jax.ShapeDtypeStruct((NP, GC_OUT), jnp.float32),
        ],
    )


def _comb_body(nf_ref, a0_ref, a1_ref, y_ref, dis_ref, Wc_ref, bc_ref,
               bgc_ref, out_ref):
    dis = dis_ref[...]
    agg = a0_ref[...] + a1_ref[...] - y_ref[...]
    gc = agg * dis + bgc_ref[...]
    nf = nf_ref[...]
    pre = (_dot(nf, Wc_ref[0:H2, :]) + _dot(gc, Wc_ref[H2:H2 + GC_OUT, :])
           + bc_ref[...])
    out_ref[...] = _elu(pre)


def _make_comb_call():
    return pl.pallas_call(
        _comb_body,
        grid=(GRID,),
        in_specs=[
            pl.BlockSpec((BN, H2), lambda i: (i, 0)),
            pl.BlockSpec((BN, GC_OUT), lambda i: (i, 0)),
            pl.BlockSpec((BN, GC_OUT), lambda i: (i, 0)),
            pl.BlockSpec((BN, GC_OUT), lambda i: (i, 0)),
            pl.BlockSpec((BN, 1), lambda i: (i, 0)),
            pl.BlockSpec((H2 + GC_OUT, GC_OUT), lambda i: (0, 0)),
            pl.BlockSpec((1, GC_OUT), lambda i: (0, 0)),
            pl.BlockSpec((1, GC_OUT), lambda i: (0, 0)),
        ],
        out_specs=pl.BlockSpec((BN, GC_OUT), lambda i: (i, 0)),
        out_shape=jax.ShapeDtypeStruct((N, GC_OUT), jnp.float32),
    )


def kernel(feats, edges, batch, W1, b1, W2, b2, Wgc, bgc, Wc, bc):
    src = edges[0]
    dst = edges[1]

    deg_parts = _make_deg_call()(dst)
    dis = _make_dis_call()(deg_parts)
    nfeats, y = _make_mlp_call()(
        feats, dis, W1, b1.reshape(1, -1), W2, b2.reshape(1, -1), Wgc)
    a0, a1 = _make_agg_call()(src, dst, y)
    out = _make_comb_call()(nfeats, a0, a1, y, dis, Wc,
                            bc.reshape(1, -1), bgc.reshape(1, -1))
    return (out, edges, batch)
